# baseline (speedup 1.0000x reference)
import functools

import jax
import jax.numpy as jnp
from jax import lax
from jax.experimental import pallas as pl
from jax.experimental.pallas import tpu as pltpu
from jax.experimental.pallas import tpu_sc as plsc

B = 16384
D_OUT = 151
NC, NS, NL = 2, 16, 16
NW = NC * NS
B_PER_W = B // NW
HALF = B_PER_W // 2
ACC_WORDS = B_PER_W * D_OUT
ROW_OFF = (0, 128, 384)


def _body(idx_hbm, wb_hbm, ws_hbm, wt_hbm, wd_hbm, out_hbm,
          idx_v, dir_v, tab_v, acc_v,
          sem0, sem1):
    wid = lax.axis_index("s") * NC + lax.axis_index("c")
    base = wid * B_PER_W

    pltpu.sync_copy(idx_hbm.at[pl.ds(base * 4, B_PER_W * 4)], idx_v)
    pltpu.sync_copy(wd_hbm, dir_v)
    pltpu.sync_copy(wb_hbm, tab_v.at[pl.ds(0, 128)])
    pltpu.sync_copy(ws_hbm, tab_v.at[pl.ds(128, 256)])
    pltpu.sync_copy(wt_hbm, tab_v.at[pl.ds(384, 128)])

    iota = lax.iota(jnp.int32, NL)
    cols = [iota, iota + 16, iota + 32]

    def repack_half(lo, hi):
        @plsc.parallel_loop(lo, hi, unroll=4)
        def _rows(b):
            bq = jnp.full((NL,), b * 4, jnp.int32)
            d = jnp.full((NL,), b * D_OUT, jnp.int32) + iota
            for k in range(3):
                tk = plsc.load_gather(idx_v, [bq + k]) + ROW_OFF[k]
                for c in range(3):
                    v = plsc.load_gather(tab_v, [tk, cols[c]])
                    plsc.store_scatter(acc_v, [d + (k * 50 + c * 16)], v)

        @plsc.parallel_loop(lo // NL, hi // NL, unroll=2)
        def _tails(j):
            rq = (j * NL + iota) * 4
            dg = (j * NL + iota) * D_OUT
            for k in range(3):
                tkv = plsc.load_gather(idx_v, [rq + k]) + ROW_OFF[k]
                for c in (48, 49):
                    v = plsc.load_gather(
                        tab_v, [tkv, jnp.full((NL,), c, jnp.int32)])
                    plsc.store_scatter(acc_v, [dg + (k * 50 + c)], v)
            dvals = plsc.load_gather(dir_v, [plsc.load_gather(idx_v, [rq + 3])])
            plsc.store_scatter(acc_v, [dg + 150], dvals)

    repack_half(0, HALF)
    cp0 = pltpu.async_copy(
        acc_v.at[pl.ds(0, HALF * D_OUT)],
        out_hbm.at[pl.ds(base * D_OUT, HALF * D_OUT)], sem0)

    repack_half(HALF, B_PER_W)
    cp1 = pltpu.async_copy(
        acc_v.at[pl.ds(HALF * D_OUT, HALF * D_OUT)],
        out_hbm.at[pl.ds(base * D_OUT + HALF * D_OUT, HALF * D_OUT)], sem1)
    cp0.wait()
    cp1.wait()


@jax.jit
def _run(idx_flat, wb, ws, wt, dir16):
    mesh = plsc.VectorSubcoreMesh(core_axis_name="c", subcore_axis_name="s")
    out_flat = pl.kernel(
        _body,
        out_type=jax.ShapeDtypeStruct((B * D_OUT,), jnp.float32),
        mesh=mesh,
        scratch_types=[
            pltpu.VMEM((B_PER_W * 4,), jnp.int32),
            pltpu.VMEM((NL,), jnp.float32),
            pltpu.VMEM((512, 50), jnp.float32),
            pltpu.VMEM((ACC_WORDS,), jnp.float32),
            pltpu.SemaphoreType.DMA,
            pltpu.SemaphoreType.DMA,
        ],
        compiler_params=pltpu.CompilerParams(
            use_tc_tiling_on_sc=False, needs_layout_passes=False),
    )(idx_flat, wb, ws, wt, dir16)
    return out_flat.reshape(B, D_OUT)


def kernel(cat_tensor, W_bus_id, W_station_id, W_time_period, W_direction):
    idx_flat = cat_tensor.astype(jnp.int32).reshape(-1)
    dir16 = jnp.pad(W_direction[:, 0], (0, NL - W_direction.shape[0]))
    return _run(idx_flat, W_bus_id, W_station_id, W_time_period, dir16)
